# baseline (speedup 1.0000x reference)
import functools

import jax
import jax.numpy as jnp
from jax import lax
from jax.experimental import pallas as pl
from jax.experimental.pallas import tpu as pltpu
from jax.experimental.pallas import tpu_sc as plsc

N = 100000
D = 128
K = 5
B = 16384
G = 3 * B
NW = 32
CH = 128
NCH_A = B // NW // CH
NCH = 3 * NCH_A
NB = 3


def _sc_gather(state, last_t, idx_s, idx_d, idx_n):
    mesh = plsc.VectorSubcoreMesh(core_axis_name="c", subcore_axis_name="s")

    @functools.partial(
        pl.kernel,
        out_type=(
            jax.ShapeDtypeStruct((G, D), jnp.float32),
            jax.ShapeDtypeStruct((3, NW, NCH_A, CH), jnp.float32),
        ),
        mesh=mesh,
        scratch_types=[
            pltpu.VMEM((NCH, CH), jnp.int32),
            pltpu.VMEM((NB, CH, D), jnp.float32),
            pltpu.VMEM((NCH, CH), jnp.float32),
            pltpu.SemaphoreType.DMA,
            pltpu.SemaphoreType.DMA,
            pltpu.SemaphoreType.DMA,
            pltpu.SemaphoreType.DMA,
        ],
    )
    def k(state_hbm, lastt_hbm, ids_hbm, idd_hbm, idn_hbm, rows_out, lt_out,
          idx_v, rows_v, lt_v, sem_lt, s0, s1, s2):
        sems = (s0, s1, s2)
        wid = lax.axis_index("s") * 2 + lax.axis_index("c")
        pltpu.sync_copy(ids_hbm.at[wid], idx_v.at[pl.ds(0 * NCH_A, NCH_A)])
        pltpu.sync_copy(idd_hbm.at[wid], idx_v.at[pl.ds(1 * NCH_A, NCH_A)])
        pltpu.sync_copy(idn_hbm.at[wid], idx_v.at[pl.ds(2 * NCH_A, NCH_A)])
        lt_cps = [
            pltpu.async_copy(lastt_hbm.at[idx_v.at[j]], lt_v.at[j], sem_lt)
            for j in range(NCH)
        ]
        gcp = [None] * NCH
        for j in range(NB - 1):
            gcp[j] = pltpu.async_copy(
                state_hbm.at[idx_v.at[j]], rows_v.at[j % NB], sems[j % NB])
        for j in range(NCH):
            nxt = j + NB - 1
            if nxt < NCH:
                gcp[nxt] = pltpu.async_copy(
                    state_hbm.at[idx_v.at[nxt]], rows_v.at[nxt % NB],
                    sems[nxt % NB])
            gcp[j].wait()
            a, jj = divmod(j, NCH_A)
            base = a * B + wid * (B // NW) + jj * CH
            pltpu.sync_copy(rows_v.at[j % NB], rows_out.at[pl.ds(base, CH)])
        for cp in lt_cps:
            cp.wait()
        for a in range(3):
            pltpu.sync_copy(lt_v.at[pl.ds(a * NCH_A, NCH_A)],
                            lt_out.at[a, wid])

    return k(state, last_t, idx_s, idx_d, idx_n)


_RB = 2048
_NBLK = G // _RB
_BPA = B // _RB


def _tc_body(ld_ref, rows_ref, lt_ref, t_ref, w1t_ref, b1_ref, w2t_ref,
             b2_ref, out_ref):
    ltv = lt_ref[0, 0, :]
    tv = t_ref[0, 0, :]
    dt = jnp.maximum(tv - ltv, 0.0)
    ld = ld_ref[0, 0]
    decay = jnp.log1p(jnp.exp(jnp.full((128,), ld, jnp.float32)))[0]
    gate = jnp.exp(-decay * dt)
    x = rows_ref[...].astype(jnp.bfloat16)
    yt = lax.dot_general(w1t_ref[...], x, (((1,), (1,)), ((), ())),
                         preferred_element_type=jnp.float32)
    ht = jnp.maximum(yt * gate[None, :] + b1_ref[...], 0.0)
    logits_t = jnp.dot(w2t_ref[...], ht.astype(jnp.bfloat16),
                       preferred_element_type=jnp.float32) + b2_ref[...]
    m = jnp.max(logits_t, axis=0, keepdims=True)
    e = jnp.exp(logits_t - m)
    p_t = e / jnp.sum(e, axis=0, keepdims=True)
    out_ref[...] = p_t.T[None]


def _tc_mlp(rows, lt_g, t, log_decay, W1, b1, W2, b2, interpret=False):
    lt3 = lt_g.reshape(_NBLK, 1, _RB)
    t3 = t.reshape(_BPA, 1, _RB)
    ld = jnp.reshape(log_decay, (1, 1))
    w1t = W1.T.astype(jnp.bfloat16)
    w2t = W2.T.astype(jnp.bfloat16)
    b1c = b1.reshape(D, 1)
    b2c = b2.reshape(K, 1)
    return pl.pallas_call(
        _tc_body,
        grid=(_NBLK,),
        in_specs=[
            pl.BlockSpec(memory_space=pltpu.SMEM),
            pl.BlockSpec((_RB, D), lambda i: (i, 0)),
            pl.BlockSpec((1, 1, _RB), lambda i: (i, 0, 0)),
            pl.BlockSpec((1, 1, _RB), lambda i: (i % _BPA, 0, 0)),
            pl.BlockSpec((D, D), lambda i: (0, 0)),
            pl.BlockSpec((D, 1), lambda i: (0, 0)),
            pl.BlockSpec((K, D), lambda i: (0, 0)),
            pl.BlockSpec((K, 1), lambda i: (0, 0)),
        ],
        out_specs=pl.BlockSpec((1, _RB, K), lambda i: (i // _BPA, i % _BPA, 0)),
        out_shape=jax.ShapeDtypeStruct((3, B, K), jnp.float32),
        interpret=interpret,
    )(ld, rows, lt3, t3, w1t, b1c, w2t, b2c)


def kernel(source_nodes, destination_nodes, negative_nodes, edge_times,
           edge_idxs, state, last_t, log_decay, W1, b1, W2, b2):
    shp = (NW, NCH_A, CH)
    rows, lt_g = _sc_gather(state, last_t,
                            source_nodes.reshape(shp),
                            destination_nodes.reshape(shp),
                            negative_nodes.reshape(shp))
    out = _tc_mlp(rows, lt_g.reshape(G), edge_times, log_decay,
                  W1, b1, W2, b2)
    return (out[0], out[1], out[2])


# speedup vs baseline: 1.2464x; 1.2464x over previous
import functools

import jax
import jax.numpy as jnp
from jax import lax
from jax.experimental import pallas as pl
from jax.experimental.pallas import tpu as pltpu
from jax.experimental.pallas import tpu_sc as plsc

N = 100000
D = 128
K = 5
B = 16384
NW = 32
PER_W = B // NW
CH = 128
NCH = PER_W // CH
NB = 3


def _sc_gather_one(state, last_t, idx):
    mesh = plsc.VectorSubcoreMesh(core_axis_name="c", subcore_axis_name="s")

    @functools.partial(
        pl.kernel,
        out_type=(
            jax.ShapeDtypeStruct((B, D), jnp.float32),
            jax.ShapeDtypeStruct((B,), jnp.float32),
        ),
        mesh=mesh,
        scratch_types=[
            pltpu.VMEM((PER_W,), jnp.int32),
            pltpu.VMEM((NB, CH, D), jnp.float32),
            pltpu.VMEM((PER_W,), jnp.float32),
            pltpu.SemaphoreType.DMA,
            pltpu.SemaphoreType.DMA,
            pltpu.SemaphoreType.DMA,
            pltpu.SemaphoreType.DMA,
        ],
    )
    def k(state_hbm, lastt_hbm, idx_hbm, rows_out, lt_out, idx_v, rows_v,
          lt_v, sem_lt, s0, s1, s2):
        sems = (s0, s1, s2)
        wid = lax.axis_index("s") * 2 + lax.axis_index("c")
        base_w = wid * PER_W
        pltpu.sync_copy(idx_hbm.at[pl.ds(base_w, PER_W)], idx_v)
        lt_cps = [
            pltpu.async_copy(lastt_hbm.at[idx_v.at[pl.ds(j * CH, CH)]],
                             lt_v.at[pl.ds(j * CH, CH)], sem_lt)
            for j in range(NCH)
        ]
        gcp = [None] * NCH
        for j in range(NB - 1):
            gcp[j] = pltpu.async_copy(
                state_hbm.at[idx_v.at[pl.ds(j * CH, CH)]],
                rows_v.at[j % NB], sems[j % NB])
        for j in range(NCH):
            nxt = j + NB - 1
            if nxt < NCH:
                gcp[nxt] = pltpu.async_copy(
                    state_hbm.at[idx_v.at[pl.ds(nxt * CH, CH)]],
                    rows_v.at[nxt % NB], sems[nxt % NB])
            gcp[j].wait()
            pltpu.sync_copy(rows_v.at[j % NB],
                            rows_out.at[pl.ds(base_w + j * CH, CH)])
        for cp in lt_cps:
            cp.wait()
        pltpu.sync_copy(lt_v, lt_out.at[pl.ds(base_w, PER_W)])

    return k(state, last_t, idx)


_RB = 2048
_NBLK = B // _RB


def _tc_body(ld_ref, rows_ref, lt_ref, t_ref, w1t_ref, b1_ref, w2t_ref,
             b2_ref, out_ref):
    ltv = lt_ref[0, 0, :]
    tv = t_ref[0, 0, :]
    dt = jnp.maximum(tv - ltv, 0.0)
    ld = ld_ref[0, 0]
    decay = jnp.log1p(jnp.exp(jnp.full((128,), ld, jnp.float32)))[0]
    gate = jnp.exp(-decay * dt)
    x = rows_ref[...].astype(jnp.bfloat16)
    yt = lax.dot_general(w1t_ref[...], x, (((1,), (1,)), ((), ())),
                         preferred_element_type=jnp.float32)
    ht = jnp.maximum(yt * gate[None, :] + b1_ref[...], 0.0)
    logits_t = jnp.dot(w2t_ref[...], ht.astype(jnp.bfloat16),
                       preferred_element_type=jnp.float32) + b2_ref[...]
    m = jnp.max(logits_t, axis=0, keepdims=True)
    e = jnp.exp(logits_t - m)
    p_t = e / jnp.sum(e, axis=0, keepdims=True)
    out_ref[...] = p_t.T


def _tc_mlp_one(rows, lt_g, t, ld, w1t, b1c, w2t, b2c, interpret=False):
    lt3 = lt_g.reshape(_NBLK, 1, _RB)
    t3 = t.reshape(_NBLK, 1, _RB)
    return pl.pallas_call(
        _tc_body,
        grid=(_NBLK,),
        in_specs=[
            pl.BlockSpec(memory_space=pltpu.SMEM),
            pl.BlockSpec((_RB, D), lambda i: (i, 0)),
            pl.BlockSpec((1, 1, _RB), lambda i: (i, 0, 0)),
            pl.BlockSpec((1, 1, _RB), lambda i: (i, 0, 0)),
            pl.BlockSpec((D, D), lambda i: (0, 0)),
            pl.BlockSpec((D, 1), lambda i: (0, 0)),
            pl.BlockSpec((K, D), lambda i: (0, 0)),
            pl.BlockSpec((K, 1), lambda i: (0, 0)),
        ],
        out_specs=pl.BlockSpec((_RB, K), lambda i: (i, 0)),
        out_shape=jax.ShapeDtypeStruct((B, K), jnp.float32),
        interpret=interpret,
    )(ld, rows, lt3, t3, w1t, b1c, w2t, b2c)


def kernel(source_nodes, destination_nodes, negative_nodes, edge_times,
           edge_idxs, state, last_t, log_decay, W1, b1, W2, b2):
    ld = jnp.reshape(log_decay, (1, 1))
    w1t = W1.T.astype(jnp.bfloat16)
    w2t = W2.T.astype(jnp.bfloat16)
    b1c = b1.reshape(D, 1)
    b2c = b2.reshape(K, 1)
    outs = []
    for idx in (source_nodes, destination_nodes, negative_nodes):
        rows, lt_g = _sc_gather_one(state, last_t, idx)
        outs.append(_tc_mlp_one(rows, lt_g, edge_times, ld,
                                w1t, b1c, w2t, b2c))
    return tuple(outs)


# speedup vs baseline: 1.7093x; 1.3714x over previous
import functools

import jax
import jax.numpy as jnp
from jax import lax
from jax.experimental import pallas as pl
from jax.experimental.pallas import tpu as pltpu
from jax.experimental.pallas import tpu_sc as plsc

N = 100000
D = 128
K = 5
B = 16384
NW = 32
PER_W = B // NW
CH = 128
NCH = PER_W // CH
NB = 3


def _sc_gather_one(state, last_t, idx):
    mesh = plsc.VectorSubcoreMesh(core_axis_name="c", subcore_axis_name="s")

    @functools.partial(
        pl.kernel,
        out_type=(
            jax.ShapeDtypeStruct((B, D), jnp.float32),
            jax.ShapeDtypeStruct((B,), jnp.float32),
        ),
        mesh=mesh,
        scratch_types=[
            pltpu.VMEM((PER_W,), jnp.int32),
            pltpu.VMEM((NB, CH, D), jnp.float32),
            pltpu.VMEM((PER_W,), jnp.float32),
            pltpu.SemaphoreType.DMA,
            pltpu.SemaphoreType.DMA,
            pltpu.SemaphoreType.DMA,
            pltpu.SemaphoreType.DMA,
        ],
    )
    def k(state_hbm, lastt_hbm, idx_hbm, rows_out, lt_out, idx_v, rows_v,
          lt_v, sem_lt, s0, s1, s2):
        sems = (s0, s1, s2)
        wid = lax.axis_index("s") * 2 + lax.axis_index("c")
        base_w = wid * PER_W
        pltpu.sync_copy(idx_hbm.at[pl.ds(base_w, PER_W)], idx_v)
        lt_cps = [
            pltpu.async_copy(lastt_hbm.at[idx_v.at[pl.ds(j * CH, CH)]],
                             lt_v.at[pl.ds(j * CH, CH)], sem_lt)
            for j in range(NCH)
        ]
        gcp = [None] * NCH
        for j in range(NB - 1):
            gcp[j] = pltpu.async_copy(
                state_hbm.at[idx_v.at[pl.ds(j * CH, CH)]],
                rows_v.at[j % NB], sems[j % NB])
        for j in range(NCH):
            nxt = j + NB - 1
            if nxt < NCH:
                gcp[nxt] = pltpu.async_copy(
                    state_hbm.at[idx_v.at[pl.ds(nxt * CH, CH)]],
                    rows_v.at[nxt % NB], sems[nxt % NB])
            gcp[j].wait()
            pltpu.sync_copy(rows_v.at[j % NB],
                            rows_out.at[pl.ds(base_w + j * CH, CH)])
        for cp in lt_cps:
            cp.wait()
        pltpu.sync_copy(lt_v, lt_out.at[pl.ds(base_w, PER_W)])

    return k(state, last_t, idx)


_RB = 2048
_NBLK = B // _RB


def _tc_body(ld_ref, rows_ref, lt_ref, t_ref, w1t_ref, b1_ref, w2t_ref,
             b2_ref, out_ref):
    ltv = lt_ref[0, 0, :]
    tv = t_ref[0, 0, :]
    dt = jnp.maximum(tv - ltv, 0.0)
    ld = ld_ref[0, 0]
    decay = jnp.log1p(jnp.exp(jnp.full((128,), ld, jnp.float32)))[0]
    gate = jnp.exp(-decay * dt)
    x = rows_ref[...].astype(jnp.bfloat16)
    yt = lax.dot_general(w1t_ref[...], x, (((1,), (1,)), ((), ())),
                         preferred_element_type=jnp.float32)
    ht = jnp.maximum(yt * gate[None, :] + b1_ref[...], 0.0)
    logits_t = jnp.dot(w2t_ref[...], ht.astype(jnp.bfloat16),
                       preferred_element_type=jnp.float32) + b2_ref[...]
    m = jnp.max(logits_t, axis=0, keepdims=True)
    e = jnp.exp(logits_t - m)
    out_ref[...] = e / jnp.sum(e, axis=0, keepdims=True)


def _tc_mlp_one(rows, lt_g, t, ld, w1t, b1c, w2t, b2c, interpret=False):
    lt3 = lt_g.reshape(_NBLK, 1, _RB)
    t3 = t.reshape(_NBLK, 1, _RB)
    return pl.pallas_call(
        _tc_body,
        grid=(_NBLK,),
        in_specs=[
            pl.BlockSpec(memory_space=pltpu.SMEM),
            pl.BlockSpec((_RB, D), lambda i: (i, 0)),
            pl.BlockSpec((1, 1, _RB), lambda i: (i, 0, 0)),
            pl.BlockSpec((1, 1, _RB), lambda i: (i, 0, 0)),
            pl.BlockSpec((D, D), lambda i: (0, 0)),
            pl.BlockSpec((D, 1), lambda i: (0, 0)),
            pl.BlockSpec((K, D), lambda i: (0, 0)),
            pl.BlockSpec((K, 1), lambda i: (0, 0)),
        ],
        out_specs=pl.BlockSpec((K, _RB), lambda i: (0, i)),
        out_shape=jax.ShapeDtypeStruct((K, B), jnp.float32),
        interpret=interpret,
    )(ld, rows, lt3, t3, w1t, b1c, w2t, b2c)


def kernel(source_nodes, destination_nodes, negative_nodes, edge_times,
           edge_idxs, state, last_t, log_decay, W1, b1, W2, b2):
    ld = jnp.reshape(log_decay, (1, 1))
    w1t = W1.T.astype(jnp.bfloat16)
    w2t = W2.T.astype(jnp.bfloat16)
    b1c = b1.reshape(D, 1)
    b2c = b2.reshape(K, 1)
    outs = []
    for idx in (source_nodes, destination_nodes, negative_nodes):
        rows, lt_g = _sc_gather_one(state, last_t, idx)
        outs.append(_tc_mlp_one(rows, lt_g, edge_times, ld,
                                w1t, b1c, w2t, b2c).T)
    return tuple(outs)


# speedup vs baseline: 1.7140x; 1.0028x over previous
import functools

import jax
import jax.numpy as jnp
from jax import lax
from jax.experimental import pallas as pl
from jax.experimental.pallas import tpu as pltpu
from jax.experimental.pallas import tpu_sc as plsc

N = 100000
D = 128
K = 5
B = 16384
NW = 32
PER_W = B // NW
CH = 128
NCH = PER_W // CH
NB = 3


def _sc_gather_one(state, last_t, idx):
    mesh = plsc.VectorSubcoreMesh(core_axis_name="c", subcore_axis_name="s")

    @functools.partial(
        pl.kernel,
        out_type=(
            jax.ShapeDtypeStruct((B, D), jnp.float32),
            jax.ShapeDtypeStruct((B,), jnp.float32),
        ),
        mesh=mesh,
        scratch_types=[
            pltpu.VMEM((PER_W,), jnp.int32),
            pltpu.VMEM((NB, CH, D), jnp.float32),
            pltpu.VMEM((PER_W,), jnp.float32),
            pltpu.SemaphoreType.DMA,
            pltpu.SemaphoreType.DMA,
            pltpu.SemaphoreType.DMA,
            pltpu.SemaphoreType.DMA,
        ],
    )
    def k(state_hbm, lastt_hbm, idx_hbm, rows_out, lt_out, idx_v, rows_v,
          lt_v, sem_lt, s0, s1, s2):
        sems = (s0, s1, s2)
        wid = lax.axis_index("s") * 2 + lax.axis_index("c")
        base_w = wid * PER_W
        pltpu.sync_copy(idx_hbm.at[pl.ds(base_w, PER_W)], idx_v)
        lt_cps = [
            pltpu.async_copy(lastt_hbm.at[idx_v.at[pl.ds(j * CH, CH)]],
                             lt_v.at[pl.ds(j * CH, CH)], sem_lt)
            for j in range(NCH)
        ]
        gcp = [None] * NCH
        for j in range(NB - 1):
            gcp[j] = pltpu.async_copy(
                state_hbm.at[idx_v.at[pl.ds(j * CH, CH)]],
                rows_v.at[j % NB], sems[j % NB])
        for j in range(NCH):
            nxt = j + NB - 1
            if nxt < NCH:
                gcp[nxt] = pltpu.async_copy(
                    state_hbm.at[idx_v.at[pl.ds(nxt * CH, CH)]],
                    rows_v.at[nxt % NB], sems[nxt % NB])
            gcp[j].wait()
            pltpu.sync_copy(rows_v.at[j % NB],
                            rows_out.at[pl.ds(base_w + j * CH, CH)])
        for cp in lt_cps:
            cp.wait()
        pltpu.sync_copy(lt_v, lt_out.at[pl.ds(base_w, PER_W)])

    return k(state, last_t, idx)


_RB = 2048
_NBLK = B // _RB


def _tc_body(ld_ref, rows_ref, lt_ref, t_ref, w1t_ref, b1_ref, w2t_ref,
             b2_ref, out_ref):
    ltv = lt_ref[...]
    tv = t_ref[...]
    dt = jnp.maximum(tv - ltv, 0.0)
    ld = ld_ref[0, 0]
    decay = jnp.log1p(jnp.exp(jnp.full((128,), ld, jnp.float32)))[0]
    gate = jnp.exp(-decay * dt)
    x = rows_ref[...].astype(jnp.bfloat16)
    yt = lax.dot_general(w1t_ref[...], x, (((1,), (1,)), ((), ())),
                         preferred_element_type=jnp.float32)
    ht = jnp.maximum(yt * gate[None, :] + b1_ref[...], 0.0)
    logits_t = jnp.dot(w2t_ref[...], ht.astype(jnp.bfloat16),
                       preferred_element_type=jnp.float32) + b2_ref[...]
    m = jnp.max(logits_t, axis=0, keepdims=True)
    e = jnp.exp(logits_t - m)
    out_ref[...] = e / jnp.sum(e, axis=0, keepdims=True)


def _tc_mlp_one(rows, lt_g, t, ld, w1t, b1c, w2t, b2c, interpret=False):
    return pl.pallas_call(
        _tc_body,
        grid=(_NBLK,),
        in_specs=[
            pl.BlockSpec(memory_space=pltpu.SMEM),
            pl.BlockSpec((_RB, D), lambda i: (i, 0)),
            pl.BlockSpec((_RB,), lambda i: (i,)),
            pl.BlockSpec((_RB,), lambda i: (i,)),
            pl.BlockSpec((D, D), lambda i: (0, 0)),
            pl.BlockSpec((D, 1), lambda i: (0, 0)),
            pl.BlockSpec((K, D), lambda i: (0, 0)),
            pl.BlockSpec((K, 1), lambda i: (0, 0)),
        ],
        out_specs=pl.BlockSpec((K, _RB), lambda i: (0, i)),
        out_shape=jax.ShapeDtypeStruct((K, B), jnp.float32),
        interpret=interpret,
    )(ld, rows, lt_g, t, w1t, b1c, w2t, b2c)


def kernel(source_nodes, destination_nodes, negative_nodes, edge_times,
           edge_idxs, state, last_t, log_decay, W1, b1, W2, b2):
    ld = jnp.reshape(log_decay, (1, 1))
    w1t = W1.T.astype(jnp.bfloat16)
    w2t = W2.T.astype(jnp.bfloat16)
    b1c = b1.reshape(D, 1)
    b2c = b2.reshape(K, 1)
    outs = []
    for idx in (source_nodes, destination_nodes, negative_nodes):
        rows, lt_g = _sc_gather_one(state, last_t, idx)
        outs.append(_tc_mlp_one(rows, lt_g, edge_times, ld,
                                w1t, b1c, w2t, b2c).T)
    return tuple(outs)
